# baseline (speedup 1.0000x reference)
import functools

import jax
import jax.numpy as jnp
from jax import lax
from jax.experimental import pallas as pl
from jax.experimental.pallas import tpu as pltpu
from jax.experimental.pallas import tpu_sc as plsc

B = 16384
F = 64
NC = 2
NS = 16
NW = NC * NS
BPW = B // NW

def _dyn_gather(x, idx):
    return lax.gather(
        x, idx[:, None],
        dimension_numbers=lax.GatherDimensionNumbers(
            offset_dims=(), collapsed_slice_dims=(0,), start_index_map=(0,)),
        slice_sizes=(1,),
        mode=lax.GatherScatterMode.PROMISE_IN_BOUNDS)


@functools.lru_cache(maxsize=1)
def _make_gather3():
    mesh = plsc.VectorSubcoreMesh(core_axis_name="c", subcore_axis_name="s")
    NE = BPW * F

    @functools.partial(
        pl.kernel,
        mesh=mesh,
        out_type=[
            jax.ShapeDtypeStruct((B * F,), jnp.float32),
            jax.ShapeDtypeStruct((B * F,), jnp.float32),
            jax.ShapeDtypeStruct((B * F,), jnp.float32),
        ],
        scratch_types=[
            pltpu.VMEM((BPW,), jnp.int32),
            pltpu.VMEM((NE,), jnp.int32),
            pltpu.VMEM((NE,), jnp.float32),
            pltpu.SemaphoreType.DMA,
        ],
    )
    def _gather3(user_t, movie_t, genre_t, uidx, midx, gidx,
                 out_u, out_m, out_g, iv, ev, rows, sem):
        wid = lax.axis_index("s") * NC + lax.axis_index("c")
        base = wid * BPW
        lanes = lax.iota(jnp.int32, 16)

        def one_table(table, idx, out):
            pltpu.sync_copy(idx.at[pl.ds(base, BPW)], iv)

            def grp(i, _):
                g16 = iv[pl.ds(i * 16, 16)] * F
                for r in range(16):
                    b = _dyn_gather(g16, jnp.full((16,), r, jnp.int32)) + lanes
                    for c in range(F // 16):
                        ev[pl.ds(i * 16 * F + r * F + c * 16, 16)] = (
                            b + c * 16)
                return 0

            lax.fori_loop(0, BPW // 16, grp, 0)
            pltpu.async_copy(table.at[ev], rows, sem).wait()
            pltpu.sync_copy(rows, out.at[pl.ds(base * F, NE)])

        one_table(user_t, uidx, out_u)
        one_table(movie_t, midx, out_m)
        one_table(genre_t, gidx, out_g)

    return _gather3


BT = 2048


def _mlp_body(ue, me, ge, w1u, w1m, w1g, b1, w2, b2, w3, b3, out):
    x = (jnp.dot(ue[...], w1u[...], preferred_element_type=jnp.float32)
         + jnp.dot(me[...], w1m[...], preferred_element_type=jnp.float32)
         + jnp.dot(ge[...], w1g[...], preferred_element_type=jnp.float32)
         + b1[...])
    x = jnp.maximum(x, 0.0)
    x = jnp.maximum(
        jnp.dot(x, w2[...], preferred_element_type=jnp.float32) + b2[...], 0.0)
    out[...] = jnp.dot(x, w3[...], preferred_element_type=jnp.float32) + b3[...]


def _mlp(ue, me, ge, w1u, w1m, w1g, b1, w2, b2, w3, b3, *, interpret=False):
    grid = B // BT
    full = lambda shape: pl.BlockSpec(shape, lambda i: (0, 0))
    return pl.pallas_call(
        _mlp_body,
        grid=(grid,),
        in_specs=[
            pl.BlockSpec((BT, F), lambda i: (i, 0)),
            pl.BlockSpec((BT, F), lambda i: (i, 0)),
            pl.BlockSpec((BT, F), lambda i: (i, 0)),
            full((F, 128)),
            full((F, 128)),
            full((F, 128)),
            full((1, 128)),
            full((128, F)),
            full((1, F)),
            full((F, 1)),
            full((1, 1)),
        ],
        out_specs=pl.BlockSpec((BT, 1), lambda i: (i, 0)),
        out_shape=jax.ShapeDtypeStruct((B, 1), jnp.float32),
        interpret=interpret,
    )(ue, me, ge, w1u, w1m, w1g, b1, w2, b2, w3, b3)


def kernel(user, movie, genres, user_table, movie_table, genre_table,
           W1, b1, W2, b2, W3, b3):
    ue, me, ge = _make_gather3()(user_table.reshape(-1),
                                 movie_table.reshape(-1),
                                 genre_table.reshape(-1),
                                 user, movie, genres)
    ue = ue.reshape(B, F)
    me = me.reshape(B, F)
    ge = ge.reshape(B, F)
    w1u = W1[:, :F].T
    w1m = W1[:, F:2 * F].T
    w1g = W1[:, 2 * F:].T
    return _mlp(ue, me, ge, w1u, w1m, w1g,
                b1.reshape(1, 128), W2.T, b2.reshape(1, F),
                W3.T, b3.reshape(1, 1))


# speedup vs baseline: 1.8239x; 1.8239x over previous
import functools

import jax
import jax.numpy as jnp
from jax import lax
from jax.experimental import pallas as pl
from jax.experimental.pallas import tpu as pltpu
from jax.experimental.pallas import tpu_sc as plsc

B = 16384
F = 64
NC = 2
NS = 16
NW = NC * NS
BPW = B // NW
WIN = 16


@functools.lru_cache(maxsize=1)
def _make_gather3():
    mesh = plsc.VectorSubcoreMesh(core_axis_name="c", subcore_axis_name="s")

    @functools.partial(
        pl.kernel,
        mesh=mesh,
        out_type=[
            jax.ShapeDtypeStruct((B, F), jnp.float32),
            jax.ShapeDtypeStruct((B, F), jnp.float32),
            jax.ShapeDtypeStruct((B, F), jnp.float32),
        ],
        scratch_types=[
            pltpu.VMEM((BPW + 16,), jnp.int32),
            pltpu.VMEM((BPW, F), jnp.float32),
            pltpu.SemaphoreType.DMA,
        ],
    )
    def _gather3(ut, mt, gt, uidx, midx, gidx, out_u, out_m, out_g,
                 iv, rows, sem):
        wid = lax.axis_index("s") * NC + lax.axis_index("c")
        base = wid * BPW

        def one_table(table, idx, out):
            pltpu.sync_copy(idx.at[pl.ds(base, BPW)], iv.at[pl.ds(0, BPW)])

            def step(r, _):
                s = iv[pl.ds(r, 16)][0]
                pltpu.async_copy(
                    table.at[pl.ds(s, 1)], rows.at[pl.ds(r, 1)], sem)

                @pl.when(r >= WIN)
                def _():
                    pltpu.make_async_copy(
                        table.at[pl.ds(0, 1)], rows.at[pl.ds(0, 1)],
                        sem).wait()

                return 0

            lax.fori_loop(0, BPW, step, 0)
            for _ in range(WIN):
                pltpu.make_async_copy(
                    table.at[pl.ds(0, 1)], rows.at[pl.ds(0, 1)], sem).wait()
            pltpu.sync_copy(rows, out.at[pl.ds(base, BPW)])

        one_table(ut, uidx, out_u)
        one_table(mt, midx, out_m)
        one_table(gt, gidx, out_g)

    return _gather3


BT = 2048
GRID = B // BT


def _mlp_body(ue, me, ge, w1u, w1m, w1g, b1, w2, b2, w3, b3, out):
    x = (jnp.dot(ue[...], w1u[...], preferred_element_type=jnp.float32)
         + jnp.dot(me[...], w1m[...], preferred_element_type=jnp.float32)
         + jnp.dot(ge[...], w1g[...], preferred_element_type=jnp.float32)
         + b1[...])
    x = jnp.maximum(x, 0.0)
    x = jnp.maximum(
        jnp.dot(x, w2[...], preferred_element_type=jnp.float32) + b2[...], 0.0)
    out[...] = jnp.dot(x, w3[...], preferred_element_type=jnp.float32) + b3[...]


def _mlp(ue, me, ge, w1u, w1m, w1g, b1, w2, b2, w3, b3, *, interpret=False):
    full = lambda shape: pl.BlockSpec(shape, lambda i: (0, 0))
    return pl.pallas_call(
        _mlp_body,
        grid=(GRID,),
        in_specs=[
            pl.BlockSpec((BT, F), lambda i: (i, 0)),
            pl.BlockSpec((BT, F), lambda i: (i, 0)),
            pl.BlockSpec((BT, F), lambda i: (i, 0)),
            full((F, 128)),
            full((F, 128)),
            full((F, 128)),
            full((1, 128)),
            full((128, F)),
            full((1, F)),
            full((F, 1)),
            full((1, 1)),
        ],
        out_specs=pl.BlockSpec((BT, 1), lambda i: (i, 0)),
        out_shape=jax.ShapeDtypeStruct((B, 1), jnp.float32),
        interpret=interpret,
    )(ue, me, ge, w1u, w1m, w1g, b1, w2, b2, w3, b3)


def kernel(user, movie, genres, user_table, movie_table, genre_table,
           W1, b1, W2, b2, W3, b3):
    ue, me, ge = _make_gather3()(user_table, movie_table, genre_table,
                                 user, movie, genres)
    w1u = W1[:, :F].T
    w1m = W1[:, F:2 * F].T
    w1g = W1[:, 2 * F:].T
    return _mlp(ue, me, ge, w1u, w1m, w1g,
                b1.reshape(1, 128), W2.T, b2.reshape(1, F),
                W3.T, b3.reshape(1, 1))
